# baseline (speedup 1.0000x reference)
import functools

import jax
import jax.numpy as jnp
from jax import lax
from jax.experimental import pallas as pl
from jax.experimental.pallas import tpu as pltpu
from jax.experimental.pallas import tpu_sc as plsc

_B = 16384
_D = 64
_NC = 2
_NS = 16
_NW = _NC * _NS
_HP = _B // (2 * _NW)
_L = 16
_HH = 128
_CH = 50176
_RQ = 512


def _sc_body(f_hbm, t_hbm, p_hbm, o0_hbm, o1_hbm,
             idx_v, idxh_v, rows_v, fa_v, fb_v, o0_v, o1_v, sem):
    wid = lax.axis_index("s") * _NC + lax.axis_index("c")
    base_a = wid * _HP
    base_b = _B - (wid + 1) * _HP

    nb = _HH // _L
    iota = lax.iota(jnp.int32, _L)

    def rev(v):
        return lax.rev(v, (0,))

    def normpair(v0, v1):
        ss = v0 * v0 + v1 * v1
        half = ss * 0.5
        bits = lax.bitcast_convert_type(ss, jnp.int32)
        bits = 0x5F3759DF - lax.shift_right_logical(bits, 1)
        r = lax.bitcast_convert_type(bits, jnp.float32)
        for _ in range(3):
            r = r * (1.5 - half * r * r)
        denom = jnp.maximum(ss * r, 1e-12)
        return v0 / denom, v1 / denom

    for p in range(2):
        ra = base_a + p * _HH
        rb = base_b + (1 - p) * _HH

        pltpu.sync_copy(t_hbm.at[pl.ds(ra, _HH)], idx_v.at[0])
        pltpu.sync_copy(t_hbm.at[pl.ds(rb, _HH)], idx_v.at[1])

        for k in range(2):
            for o in range(_HH // _L):
                sl = pl.ds(o * _L, _L)
                tv = idx_v[k, sl]
                idxh_v[k, sl] = jnp.where(tv >= _CH, tv - _CH, tv)

        copies = [
            pltpu.async_copy(p_hbm.at[idxh_v.at[0]],
                             rows_v.at[pl.ds(0, _HH)], sem),
            pltpu.async_copy(p_hbm.at[idxh_v.at[1]],
                             rows_v.at[pl.ds(_HH, _HH)], sem),
            pltpu.async_copy(f_hbm.at[:, pl.ds(ra, _HH)], fa_v, sem),
            pltpu.async_copy(f_hbm.at[:, pl.ds(rb, _HH)], fb_v, sem),
        ]
        for c in copies:
            c.wait()

        def body(t, carry):
            tp = nb - 1 - t
            ia = t * _L + iota
            iap = tp * _L + iota
            iad = t * _L + 15 - iota
            iapd = tp * _L + 15 - iota
            def halfbase(k, blk):
                tv = idx_v[k, pl.ds(blk * _L, _L)]
                return jnp.where(tv >= _CH, _D, 0).astype(jnp.int32)
            pa_t = halfbase(0, t)
            pa_p = halfbase(0, tp)
            pb_t = halfbase(1, t)
            pb_p = halfbase(1, tp)
            pa_td, pa_pd, pb_td, pb_pd = (
                rev(pa_t), rev(pa_p), rev(pb_t), rev(pb_p))
            zero = jnp.zeros((_L,), jnp.float32)
            a00t = a00p = a0bt = a0bp = zero
            a1at = a1ap = a1bt = a1bp = zero
            for d in range(_D):
                c = (iota + d) & (_D - 1)
                ga_t = plsc.load_gather(rows_v, [ia, pa_t + c])
                ga_p = plsc.load_gather(rows_v, [iap, pa_p + c])
                gb_t = plsc.load_gather(rows_v, [_HH + ia, pb_t + c])
                gb_p = plsc.load_gather(rows_v, [_HH + iap, pb_p + c])
                fa_t = plsc.load_gather(fa_v, [c, ia])
                fa_p = plsc.load_gather(fa_v, [c, iap])
                fb_t = plsc.load_gather(fb_v, [c, ia])
                fb_p = plsc.load_gather(fb_v, [c, iap])
                gmb_t = plsc.load_gather(rows_v, [_HH + iapd, pb_pd + c])
                gmb_p = plsc.load_gather(rows_v, [_HH + iad, pb_td + c])
                gma_t = plsc.load_gather(rows_v, [iapd, pa_pd + c])
                gma_p = plsc.load_gather(rows_v, [iad, pa_td + c])
                a00t = a00t + ga_t * fa_t
                a00p = a00p + ga_p * fa_p
                a0bt = a0bt + gb_t * fb_t
                a0bp = a0bp + gb_p * fb_p
                a1at = a1at + gmb_t * fa_t
                a1ap = a1ap + gmb_p * fa_p
                a1bt = a1bt + gma_t * fb_t
                a1bp = a1bp + gma_p * fb_p
            n0, n1 = normpair(a00t, a1at)
            o0_v[pl.ds(t * _L, _L)] = n0
            o1_v[pl.ds(t * _L, _L)] = n1
            n0, n1 = normpair(a00p, a1ap)
            o0_v[pl.ds(tp * _L, _L)] = n0
            o1_v[pl.ds(tp * _L, _L)] = n1
            n0, n1 = normpair(a0bt, a1bt)
            o0_v[pl.ds(_HH + t * _L, _L)] = n0
            o1_v[pl.ds(_HH + t * _L, _L)] = n1
            n0, n1 = normpair(a0bp, a1bp)
            o0_v[pl.ds(_HH + tp * _L, _L)] = n0
            o1_v[pl.ds(_HH + tp * _L, _L)] = n1
            return carry

        lax.fori_loop(0, nb // 2, body, 0)

        pltpu.sync_copy(o0_v.at[pl.ds(0, _HH)], o0_hbm.at[pl.ds(ra, _HH)])
        pltpu.sync_copy(o0_v.at[pl.ds(_HH, _HH)], o0_hbm.at[pl.ds(rb, _HH)])
        pltpu.sync_copy(o1_v.at[pl.ds(0, _HH)], o1_hbm.at[pl.ds(ra, _HH)])
        pltpu.sync_copy(o1_v.at[pl.ds(_HH, _HH)], o1_hbm.at[pl.ds(rb, _HH)])


_sc_call = functools.partial(
    pl.kernel,
    out_type=[jax.ShapeDtypeStruct((_B,), jnp.float32),
              jax.ShapeDtypeStruct((_B,), jnp.float32)],
    mesh=plsc.VectorSubcoreMesh(core_axis_name="c", subcore_axis_name="s"),
    compiler_params=pltpu.CompilerParams(needs_layout_passes=False),
    scratch_types=[
        pltpu.VMEM((2, _HH), jnp.int32),
        pltpu.VMEM((2, _HH), jnp.int32),
        pltpu.VMEM((2 * _HH, 2 * _D), jnp.float32),
        pltpu.VMEM((_D, _HH), jnp.float32),
        pltpu.VMEM((_D, _HH), jnp.float32),
        pltpu.VMEM((2 * _HH,), jnp.float32),
        pltpu.VMEM((2 * _HH,), jnp.float32),
        pltpu.SemaphoreType.DMA,
    ],
)(_sc_body)


def _repack_body(a_ref, b_ref, out_ref):
    out_ref[:, 0:_D] = a_ref[...].T
    out_ref[:, _D:2 * _D] = b_ref[...].T


def _repack(pt):
    return pl.pallas_call(
        _repack_body,
        grid=(_CH // _RQ,),
        in_specs=[
            pl.BlockSpec((_D, _RQ), lambda i: (0, i)),
            pl.BlockSpec((_D, _RQ), lambda i: (0, i + _CH // _RQ)),
        ],
        out_specs=pl.BlockSpec((_RQ, 2 * _D), lambda i: (i, 0)),
        out_shape=jax.ShapeDtypeStruct((_CH, 2 * _D), jnp.float32),
    )(pt, pt)


def kernel(f, targets, prototype):
    pairs = _repack(prototype.T)
    s0, s1 = _sc_call(f.T, targets.astype(jnp.int32), pairs)
    return jnp.stack([s0, s1], axis=-1)


# speedup vs baseline: 1.0565x; 1.0565x over previous
import functools

import jax
import jax.numpy as jnp
from jax import lax
from jax.experimental import pallas as pl
from jax.experimental.pallas import tpu as pltpu
from jax.experimental.pallas import tpu_sc as plsc

_B = 16384
_D = 64
_NC = 2
_NS = 16
_NW = _NC * _NS
_HP = _B // (2 * _NW)
_L = 16
_HH = 128


def _sc_body(f_hbm, t_hbm, p_hbm, o0_hbm, o1_hbm,
             idx_v, idxh_v, rows_v, fa_v, fb_v, o0_v, o1_v, sem):
    wid = lax.axis_index("s") * _NC + lax.axis_index("c")
    base_a = wid * _HP
    base_b = _B - (wid + 1) * _HP

    nb = _HH // _L
    iota = lax.iota(jnp.int32, _L)

    def rev(v):
        return lax.rev(v, (0,))

    def normpair(v0, v1):
        ss = v0 * v0 + v1 * v1
        half = ss * 0.5
        bits = lax.bitcast_convert_type(ss, jnp.int32)
        bits = 0x5F3759DF - lax.shift_right_logical(bits, 1)
        r = lax.bitcast_convert_type(bits, jnp.float32)
        for _ in range(3):
            r = r * (1.5 - half * r * r)
        denom = jnp.maximum(ss * r, 1e-12)
        return v0 / denom, v1 / denom

    for p in range(2):
        ra = base_a + p * _HH
        rb = base_b + (1 - p) * _HH

        pltpu.sync_copy(t_hbm.at[pl.ds(ra, _HH)], idx_v.at[0])
        pltpu.sync_copy(t_hbm.at[pl.ds(rb, _HH)], idx_v.at[1])

        for k in range(2):
            for o in range(_HH // _L):
                sl = pl.ds(o * _L, _L)
                idxh_v[k, sl] = lax.shift_right_logical(idx_v[k, sl], 1)

        copies = [
            pltpu.async_copy(p_hbm.at[idxh_v.at[0]],
                             rows_v.at[pl.ds(0, _HH)], sem),
            pltpu.async_copy(p_hbm.at[idxh_v.at[1]],
                             rows_v.at[pl.ds(_HH, _HH)], sem),
            pltpu.async_copy(f_hbm.at[:, pl.ds(ra, _HH)], fa_v, sem),
            pltpu.async_copy(f_hbm.at[:, pl.ds(rb, _HH)], fb_v, sem),
        ]
        for c in copies:
            c.wait()

        def body(t, carry):
            tp = nb - 1 - t
            ia = t * _L + iota
            iap = tp * _L + iota
            iad = t * _L + 15 - iota
            iapd = tp * _L + 15 - iota
            pa_t = lax.shift_left(idx_v[0, pl.ds(t * _L, _L)] & 1, 6)
            pa_p = lax.shift_left(idx_v[0, pl.ds(tp * _L, _L)] & 1, 6)
            pb_t = lax.shift_left(idx_v[1, pl.ds(t * _L, _L)] & 1, 6)
            pb_p = lax.shift_left(idx_v[1, pl.ds(tp * _L, _L)] & 1, 6)
            pa_td, pa_pd, pb_td, pb_pd = (
                rev(pa_t), rev(pa_p), rev(pb_t), rev(pb_p))
            zero = jnp.zeros((_L,), jnp.float32)
            a00t = a00p = a0bt = a0bp = zero
            a1at = a1ap = a1bt = a1bp = zero
            for d in range(_D):
                c = (iota + d) & (_D - 1)
                ga_t = plsc.load_gather(rows_v, [ia, pa_t + c])
                ga_p = plsc.load_gather(rows_v, [iap, pa_p + c])
                gb_t = plsc.load_gather(rows_v, [_HH + ia, pb_t + c])
                gb_p = plsc.load_gather(rows_v, [_HH + iap, pb_p + c])
                fa_t = plsc.load_gather(fa_v, [c, ia])
                fa_p = plsc.load_gather(fa_v, [c, iap])
                fb_t = plsc.load_gather(fb_v, [c, ia])
                fb_p = plsc.load_gather(fb_v, [c, iap])
                gmb_t = plsc.load_gather(rows_v, [_HH + iapd, pb_pd + c])
                gmb_p = plsc.load_gather(rows_v, [_HH + iad, pb_td + c])
                gma_t = plsc.load_gather(rows_v, [iapd, pa_pd + c])
                gma_p = plsc.load_gather(rows_v, [iad, pa_td + c])
                a00t = a00t + ga_t * fa_t
                a00p = a00p + ga_p * fa_p
                a0bt = a0bt + gb_t * fb_t
                a0bp = a0bp + gb_p * fb_p
                a1at = a1at + gmb_t * fa_t
                a1ap = a1ap + gmb_p * fa_p
                a1bt = a1bt + gma_t * fb_t
                a1bp = a1bp + gma_p * fb_p
            n0, n1 = normpair(a00t, a1at)
            o0_v[pl.ds(t * _L, _L)] = n0
            o1_v[pl.ds(t * _L, _L)] = n1
            n0, n1 = normpair(a00p, a1ap)
            o0_v[pl.ds(tp * _L, _L)] = n0
            o1_v[pl.ds(tp * _L, _L)] = n1
            n0, n1 = normpair(a0bt, a1bt)
            o0_v[pl.ds(_HH + t * _L, _L)] = n0
            o1_v[pl.ds(_HH + t * _L, _L)] = n1
            n0, n1 = normpair(a0bp, a1bp)
            o0_v[pl.ds(_HH + tp * _L, _L)] = n0
            o1_v[pl.ds(_HH + tp * _L, _L)] = n1
            return carry

        lax.fori_loop(0, nb // 2, body, 0)

        pltpu.sync_copy(o0_v.at[pl.ds(0, _HH)], o0_hbm.at[pl.ds(ra, _HH)])
        pltpu.sync_copy(o0_v.at[pl.ds(_HH, _HH)], o0_hbm.at[pl.ds(rb, _HH)])
        pltpu.sync_copy(o1_v.at[pl.ds(0, _HH)], o1_hbm.at[pl.ds(ra, _HH)])
        pltpu.sync_copy(o1_v.at[pl.ds(_HH, _HH)], o1_hbm.at[pl.ds(rb, _HH)])


_sc_call = functools.partial(
    pl.kernel,
    out_type=[jax.ShapeDtypeStruct((_B,), jnp.float32),
              jax.ShapeDtypeStruct((_B,), jnp.float32)],
    mesh=plsc.VectorSubcoreMesh(core_axis_name="c", subcore_axis_name="s"),
    compiler_params=pltpu.CompilerParams(needs_layout_passes=False),
    scratch_types=[
        pltpu.VMEM((2, _HH), jnp.int32),
        pltpu.VMEM((2, _HH), jnp.int32),
        pltpu.VMEM((2 * _HH, 2 * _D), jnp.float32),
        pltpu.VMEM((_D, _HH), jnp.float32),
        pltpu.VMEM((_D, _HH), jnp.float32),
        pltpu.VMEM((2 * _HH,), jnp.float32),
        pltpu.VMEM((2 * _HH,), jnp.float32),
        pltpu.SemaphoreType.DMA,
    ],
)(_sc_body)


def kernel(f, targets, prototype):
    pairs = prototype.reshape(prototype.shape[0] // 2, 2 * _D)
    s0, s1 = _sc_call(f.T, targets.astype(jnp.int32), pairs)
    return jnp.stack([s0, s1], axis=-1)
